# baseline (speedup 1.0000x reference)
import jax
import jax.numpy as jnp
from jax import lax
from jax.experimental import pallas as pl
from jax.experimental.pallas import tpu as pltpu
from jax.experimental.pallas import tpu_sc as plsc

_B, _S, _D = 4, 4096, 1024
_BSC = 2
_NC, _NS = 2, 16
_NW = _NC * _NS
_SW = _S // _NW
_C = 16
_NCI = _SW // _C
_J = _NCI * _BSC
_NX = 5
_LANES = 16
_DL = _D // _LANES

_BS_TC = 256


def _sc_body(x_hbm, p_hbm, o_hbm, *args):
    nb = 2 + _NX
    pbufs, xbufs = args[0:2], args[2:nb]
    psems = args[nb:nb + 2]
    xisems = args[nb + 2:nb + 2 + _NX]
    xosems = args[nb + 2 + _NX:nb + 2 + 2 * _NX]
    wid = lax.axis_index("s") * _NC + lax.axis_index("c")
    s0 = wid * _SW

    def rows(ci):
        return pl.ds(s0 + ci * _C, _C)

    def start_p(ci):
        return pltpu.async_copy(p_hbm.at[rows(ci), :],
                                pbufs[ci % 2], psems[ci % 2])

    def start_in(j):
        ci, b = divmod(j, _BSC)
        return pltpu.async_copy(x_hbm.at[b, rows(ci), :],
                                xbufs[j % _NX], xisems[j % _NX])

    def start_out(j):
        ci, b = divmod(j, _BSC)
        return pltpu.async_copy(xbufs[j % _NX],
                                o_hbm.at[b, rows(ci), :],
                                xosems[j % _NX])

    p_d = [None] * _NCI
    in_d = [None] * _J
    out_d = [None] * _J
    out_waited = set()
    p_d[0] = start_p(0)
    for k in range(min(_NX, _J)):
        in_d[k] = start_in(k)

    for j in range(_J):
        ci, b = divmod(j, _BSC)
        if b == 0:
            if ci + 1 < _NCI:
                p_d[ci + 1] = start_p(ci + 1)
            p_d[ci].wait()
        in_d[j].wait()

        xb, pb = xbufs[j % _NX], pbufs[ci % 2]

        @plsc.parallel_loop(0, _C * _DL, unroll=8)
        def _add(i):
            r = i // _DL
            sl = pl.ds((i % _DL) * _LANES, _LANES)
            plsc.addupdate(xb.at[r, sl], pb[r, sl])

        out_d[j] = start_out(j)
        nj = j + 3
        if _NX <= nj < _J:
            out_d[nj - _NX].wait()
            out_waited.add(nj - _NX)
            in_d[nj] = start_in(nj)

    for j in range(_J):
        if j not in out_waited:
            out_d[j].wait()


def _tc_body(x_ref, p_ref, o_ref):
    o_ref[...] = x_ref[...] + p_ref[...][None, :, :]


def kernel(x, p_embeddings):
    b, s, d = x.shape
    sc_run = pl.kernel(
        _sc_body,
        out_type=jax.ShapeDtypeStruct((_BSC, s, d), x.dtype),
        mesh=plsc.VectorSubcoreMesh(core_axis_name="c", subcore_axis_name="s"),
        scratch_types=(
            [pltpu.VMEM((_C, _D), jnp.float32) for _ in range(2 + _NX)]
            + [pltpu.SemaphoreType.DMA for _ in range(2 + 2 * _NX)]
        ),
    )
    out_sc = sc_run(x, p_embeddings)

    n_blocks = s // _BS_TC
    out_tc = pl.pallas_call(
        _tc_body,
        grid=(n_blocks,),
        in_specs=[
            pl.BlockSpec((b - _BSC, _BS_TC, d), lambda i: (1, i, 0)),
            pl.BlockSpec((_BS_TC, d), lambda i: (i, 0)),
        ],
        out_specs=pl.BlockSpec((b - _BSC, _BS_TC, d), lambda i: (0, i, 0)),
        out_shape=jax.ShapeDtypeStruct((b - _BSC, s, d), x.dtype),
    )(x, p_embeddings)

    return jnp.concatenate([out_sc, out_tc], axis=0)


# speedup vs baseline: 1.5193x; 1.5193x over previous
import jax
import jax.numpy as jnp
from jax import lax
from jax.experimental import pallas as pl
from jax.experimental.pallas import tpu as pltpu
from jax.experimental.pallas import tpu_sc as plsc

_B, _S, _D = 4, 4096, 1024
_NC, _NS = 2, 16
_NW = _NC * _NS
_SW = _S // _NW
_C = 16
_NCI = _SW // _C
_J = _NCI * _B
_NX = 5
_LANES = 16
_DL = _D // _LANES


def _sc_body(x_hbm, p_hbm, o_hbm, *args):
    pbufs, xbufs = args[0:2], args[2:2 + _NX]
    psems, xisems, xosems = args[7:9], args[9:9 + _NX], args[14:14 + _NX]
    wid = lax.axis_index("s") * _NC + lax.axis_index("c")
    s0 = wid * _SW

    def rows(ci):
        return pl.ds(s0 + ci * _C, _C)

    def start_p(ci):
        return pltpu.async_copy(p_hbm.at[rows(ci), :],
                                pbufs[ci % 2], psems[ci % 2])

    def start_in(j):
        ci, b = divmod(j, _B)
        return pltpu.async_copy(x_hbm.at[b, rows(ci), :],
                                xbufs[j % _NX], xisems[j % _NX])

    def start_out(j):
        ci, b = divmod(j, _B)
        return pltpu.async_copy(xbufs[j % _NX],
                                o_hbm.at[b, rows(ci), :],
                                xosems[j % _NX])

    p_d = [None] * _NCI
    in_d = [None] * _J
    out_d = [None] * _J
    out_waited = set()
    p_d[0] = start_p(0)
    for k in range(min(_NX, _J)):
        in_d[k] = start_in(k)

    for j in range(_J):
        ci, b = divmod(j, _B)
        if b == 0:
            if ci + 1 < _NCI:
                p_d[ci + 1] = start_p(ci + 1)
            p_d[ci].wait()
        in_d[j].wait()

        xb, pb = xbufs[j % _NX], pbufs[ci % 2]

        @plsc.parallel_loop(0, _C * _DL, unroll=8)
        def _add(i):
            r = i // _DL
            sl = pl.ds((i % _DL) * _LANES, _LANES)
            plsc.addupdate(xb.at[r, sl], pb[r, sl])

        out_d[j] = start_out(j)
        nj = j + 3
        if _NX <= nj < _J:
            out_d[nj - _NX].wait()
            out_waited.add(nj - _NX)
            in_d[nj] = start_in(nj)

    for j in range(_J):
        if j not in out_waited:
            out_d[j].wait()


def kernel(x, p_embeddings):
    b, s, d = x.shape
    run = pl.kernel(
        _sc_body,
        out_type=jax.ShapeDtypeStruct((b, s, d), x.dtype),
        mesh=plsc.VectorSubcoreMesh(core_axis_name="c", subcore_axis_name="s"),
        scratch_types=(
            [pltpu.VMEM((_C, _D), jnp.float32) for _ in range(2 + _NX)]
            + [pltpu.SemaphoreType.DMA for _ in range(2 + 2 * _NX)]
        ),
    )
    return run(x, p_embeddings)
